# baseline (speedup 1.0000x reference)
import functools

import jax
import jax.numpy as jnp
from jax import lax
from jax.experimental import pallas as pl
from jax.experimental.pallas import tpu as pltpu
from jax.experimental.pallas import tpu_sc as plsc

_B = 16384
_C = 1000
_EPS = 1e-08

_NC = 2
_NS = 16
_NW = _NC * _NS
_LANES = 16

_SC_ROWS = 10240
_TC_ROWS = _B - _SC_ROWS
_RPW = _SC_ROWS // _NW
_CR = 32
_NCHUNK = _RPW // _CR

_NFULL = _C // _LANES
_TAIL0 = _C - _LANES
_NTAIL = _C - _NFULL * _LANES

_ROWBLK = 2048


def _tc_body(x_ref, tgt_ref, cnt_ref):
    @pl.when(pl.program_id(0) == 0)
    def _init():
        cnt_ref[0, 0] = 0.0
        cnt_ref[0, 1] = 0.0

    x = x_ref[...]
    predf = (x >= 0.0).astype(jnp.float32)
    cnt_ref[0, 0] += jnp.sum(predf)
    col = lax.broadcasted_iota(jnp.int32, (_ROWBLK, _C), 1)
    onehotf = (col == tgt_ref[...]).astype(jnp.float32)
    cnt_ref[0, 1] += jnp.sum(predf * onehotf)


def _sc_body(x_ref, tgt_ref, out_ref, tgt_v, buf0_v, buf1_v, acc_v, sems):
    wid = lax.axis_index("s") * _NC + lax.axis_index("c")
    row0 = _TC_ROWS + wid * _RPW
    pltpu.sync_copy(tgt_ref.at[pl.ds(row0, _RPW)], tgt_v)

    lane = lax.iota(jnp.int32, _LANES)
    tailf = jnp.where(lane >= jnp.int32(_LANES - _NTAIL), 1.0, 0.0)
    bufs = (buf0_v, buf1_v)

    def _start(c):
        return pltpu.async_copy(
            x_ref.at[pl.ds(row0 + c * _CR, _CR)], bufs[c % 2], sems.at[c % 2]
        )

    copies = [None] * _NCHUNK
    copies[0] = _start(0)
    pos = jnp.zeros((_LANES,), jnp.float32)
    tp = jnp.zeros((_LANES,), jnp.float32)
    for c in range(_NCHUNK):
        if c + 1 < _NCHUNK:
            copies[c + 1] = _start(c + 1)
        copies[c].wait()
        buf = bufs[c % 2]

        def _row(r, carry):
            pos_a, tp_a = carry
            grp = tgt_v[pl.ds(c * _CR + (r // _LANES) * _LANES, _LANES)]
            idxv = jnp.broadcast_to(r % _LANES, (_LANES,)).astype(jnp.int32)
            t_splat = lax.gather(
                grp,
                idxv[:, None],
                dimension_numbers=lax.GatherDimensionNumbers(
                    offset_dims=(),
                    collapsed_slice_dims=(0,),
                    start_index_map=(0,),
                ),
                slice_sizes=(1,),
                mode=lax.GatherScatterMode.PROMISE_IN_BOUNDS,
            )
            for j in range(_NFULL + 1):
                start = j * _LANES if j < _NFULL else _TAIL0
                v = buf[r, pl.ds(start, _LANES)]
                nnf = jnp.where(v >= 0.0, 1.0, 0.0)
                if j == _NFULL:
                    nnf = nnf * tailf
                pos_a = pos_a + nnf
                col = lane + jnp.int32(start)
                eqf = jnp.where(col == t_splat, 1.0, 0.0)
                tp_a = tp_a + nnf * eqf
            return pos_a, tp_a

        pos, tp = lax.fori_loop(0, _CR, _row, (pos, tp))

    acc_v[...] = pos
    pltpu.sync_copy(acc_v, out_ref.at[0, wid])
    acc_v[...] = tp
    pltpu.sync_copy(acc_v, out_ref.at[1, wid])


@functools.cache
def _sc_scan():
    return pl.kernel(
        _sc_body,
        out_type=jax.ShapeDtypeStruct((2, _NW, _LANES), jnp.float32),
        mesh=plsc.VectorSubcoreMesh(
            core_axis_name="c",
            subcore_axis_name="s",
            num_cores=_NC,
            num_subcores=_NS,
        ),
        scratch_types=[
            pltpu.VMEM((_RPW,), jnp.int32),
            pltpu.VMEM((_CR, _C), jnp.float32),
            pltpu.VMEM((_CR, _C), jnp.float32),
            pltpu.VMEM((_LANES,), jnp.float32),
            pltpu.SemaphoreType.DMA((2,)),
        ],
    )


@jax.jit
def kernel(output, target):
    tgt = target.astype(jnp.int32)

    tc_cnt = pl.pallas_call(
        _tc_body,
        grid=(_TC_ROWS // _ROWBLK,),
        in_specs=[
            pl.BlockSpec((_ROWBLK, _C), lambda i: (i, 0)),
            pl.BlockSpec((_ROWBLK, 1), lambda i: (i, 0)),
        ],
        out_specs=pl.BlockSpec(memory_space=pltpu.SMEM),
        out_shape=jax.ShapeDtypeStruct((1, 2), jnp.float32),
    )(output, tgt.reshape(_B, 1))

    sc_parts = _sc_scan()(output, tgt)

    p_total = tc_cnt[0, 0] + jnp.sum(sc_parts[0])
    tp0 = tc_cnt[0, 1] + jnp.sum(sc_parts[1])

    fp0 = p_total - tp0
    fn0 = jnp.float32(_B) - tp0
    tn0 = jnp.float32(_B * (_C - 1)) - fp0
    eps = jnp.float32(_EPS)
    return (tp0 + eps, tn0 + eps, fp0 + eps, fn0 + eps)


# speedup vs baseline: 1.1120x; 1.1120x over previous
import functools

import jax
import jax.numpy as jnp
from jax import lax
from jax.experimental import pallas as pl
from jax.experimental.pallas import tpu as pltpu
from jax.experimental.pallas import tpu_sc as plsc

_B = 16384
_C = 1000
_EPS = 1e-08

_NC = 2
_NS = 16
_NW = _NC * _NS
_LANES = 16

_SC_ROWS = 6144
_TC_ROWS = _B - _SC_ROWS
_RPW = _SC_ROWS // _NW
_CR = 32
_NCHUNK = _RPW // _CR

_NFULL = _C // _LANES
_TAIL0 = _C - _LANES
_NTAIL = _C - _NFULL * _LANES

_ROWBLK = 2048


def _tc_body(x_ref, tgt_ref, cnt_ref):
    @pl.when(pl.program_id(0) == 0)
    def _init():
        cnt_ref[0, 0] = 0.0
        cnt_ref[0, 1] = 0.0

    x = x_ref[...]
    predf = (x >= 0.0).astype(jnp.float32)
    cnt_ref[0, 0] += jnp.sum(predf)
    col = lax.broadcasted_iota(jnp.int32, (_ROWBLK, _C), 1)
    onehotf = (col == tgt_ref[...]).astype(jnp.float32)
    cnt_ref[0, 1] += jnp.sum(predf * onehotf)


def _sc_body(x_ref, tgt_ref, out_ref, tgt_v, buf0_v, buf1_v, acc_v, sems):
    wid = lax.axis_index("s") * _NC + lax.axis_index("c")
    row0 = _TC_ROWS + wid * _RPW
    pltpu.sync_copy(tgt_ref.at[pl.ds(row0, _RPW)], tgt_v)

    lane = lax.iota(jnp.int32, _LANES)
    tailf = jnp.where(lane >= jnp.int32(_LANES - _NTAIL), 1.0, 0.0)
    bufs = (buf0_v, buf1_v)

    def _start(c):
        return pltpu.async_copy(
            x_ref.at[pl.ds(row0 + c * _CR, _CR)], bufs[c % 2], sems.at[c % 2]
        )

    copies = [None] * _NCHUNK
    copies[0] = _start(0)
    pos = jnp.zeros((_LANES,), jnp.float32)
    tp = jnp.zeros((_LANES,), jnp.float32)
    for c in range(_NCHUNK):
        if c + 1 < _NCHUNK:
            copies[c + 1] = _start(c + 1)
        copies[c].wait()
        buf = bufs[c % 2]

        def _row(r, carry):
            pos_a, tp_a = carry
            grp = tgt_v[pl.ds(c * _CR + (r // _LANES) * _LANES, _LANES)]
            idxv = jnp.broadcast_to(r % _LANES, (_LANES,)).astype(jnp.int32)
            t_splat = lax.gather(
                grp,
                idxv[:, None],
                dimension_numbers=lax.GatherDimensionNumbers(
                    offset_dims=(),
                    collapsed_slice_dims=(0,),
                    start_index_map=(0,),
                ),
                slice_sizes=(1,),
                mode=lax.GatherScatterMode.PROMISE_IN_BOUNDS,
            )
            for j in range(_NFULL + 1):
                start = j * _LANES if j < _NFULL else _TAIL0
                v = buf[r, pl.ds(start, _LANES)]
                nnf = jnp.where(v >= 0.0, 1.0, 0.0)
                if j == _NFULL:
                    nnf = nnf * tailf
                pos_a = pos_a + nnf
                col = lane + jnp.int32(start)
                eqf = jnp.where(col == t_splat, 1.0, 0.0)
                tp_a = tp_a + nnf * eqf
            return pos_a, tp_a

        pos, tp = lax.fori_loop(0, _CR, _row, (pos, tp))

    acc_v[...] = pos
    pltpu.sync_copy(acc_v, out_ref.at[0, wid])
    acc_v[...] = tp
    pltpu.sync_copy(acc_v, out_ref.at[1, wid])


@functools.cache
def _sc_scan():
    return pl.kernel(
        _sc_body,
        out_type=jax.ShapeDtypeStruct((2, _NW, _LANES), jnp.float32),
        mesh=plsc.VectorSubcoreMesh(
            core_axis_name="c",
            subcore_axis_name="s",
            num_cores=_NC,
            num_subcores=_NS,
        ),
        scratch_types=[
            pltpu.VMEM((_RPW,), jnp.int32),
            pltpu.VMEM((_CR, _C), jnp.float32),
            pltpu.VMEM((_CR, _C), jnp.float32),
            pltpu.VMEM((_LANES,), jnp.float32),
            pltpu.SemaphoreType.DMA((2,)),
        ],
    )


@jax.jit
def kernel(output, target):
    tgt = target.astype(jnp.int32)

    tc_cnt = pl.pallas_call(
        _tc_body,
        grid=(_TC_ROWS // _ROWBLK,),
        in_specs=[
            pl.BlockSpec((_ROWBLK, _C), lambda i: (i, 0)),
            pl.BlockSpec((_ROWBLK, 1), lambda i: (i, 0)),
        ],
        out_specs=pl.BlockSpec(memory_space=pltpu.SMEM),
        out_shape=jax.ShapeDtypeStruct((1, 2), jnp.float32),
    )(output, tgt.reshape(_B, 1))

    sc_parts = _sc_scan()(output, tgt)

    p_total = tc_cnt[0, 0] + jnp.sum(sc_parts[0])
    tp0 = tc_cnt[0, 1] + jnp.sum(sc_parts[1])

    fp0 = p_total - tp0
    fn0 = jnp.float32(_B) - tp0
    tn0 = jnp.float32(_B * (_C - 1)) - fp0
    eps = jnp.float32(_EPS)
    return (tp0 + eps, tn0 + eps, fp0 + eps, fn0 + eps)
